# Initial kernel scaffold; baseline (speedup 1.0000x reference)
import jax
import jax.numpy as jnp
from jax.experimental import pallas as pl


def kernel(probs, groups):
    raise NotImplementedError("write your pallas kernel here")


# speedup vs baseline: 1.4072x; 1.4072x over previous
import functools

import jax
import jax.numpy as jnp
from jax import lax
from jax.experimental import pallas as pl
from jax.experimental.pallas import tpu as pltpu
from jax.experimental.pallas import tpu_sc as plsc

_EPG = 3
_G = 5_094_400
_TOTAL = _G * _EPG

_ROWS = 39800
_COLS = 128
_BR = 200


def _sc_gather(table, idx):
    total = idx.shape[0]
    info = plsc.get_sparse_core_info()
    nc, ns = info.num_cores, info.num_subcores
    nw = nc * ns
    assert total % nw == 0
    per_w = total // nw
    ch = 16384
    n_main = per_w // ch
    tail = per_w - n_main * ch
    assert per_w % 8 == 0 and tail % 8 == 0
    mesh = plsc.VectorSubcoreMesh(core_axis_name="c", subcore_axis_name="s")

    @functools.partial(
        pl.kernel,
        mesh=mesh,
        out_type=jax.ShapeDtypeStruct((total,), table.dtype),
        scratch_types=[
            pltpu.VMEM((ch,), jnp.int32),
            pltpu.VMEM((ch,), table.dtype),
            pltpu.SemaphoreType.DMA,
        ],
    )
    def k(table_hbm, idx_hbm, out_hbm, idx_v, rows_v, sem):
        wid = lax.axis_index("s") * nc + lax.axis_index("c")
        base = wid * jnp.int32(per_w)

        for c in range(n_main):
            start = base + jnp.int32(c * ch)
            pltpu.sync_copy(idx_hbm.at[pl.ds(start, ch)], idx_v)
            pltpu.async_copy(table_hbm.at[idx_v], rows_v, sem).wait()
            pltpu.sync_copy(rows_v, out_hbm.at[pl.ds(start, ch)])
        if tail:
            start = base + jnp.int32(n_main * ch)
            iv = idx_v.at[pl.ds(0, tail)]
            rv = rows_v.at[pl.ds(0, tail)]
            pltpu.sync_copy(idx_hbm.at[pl.ds(start, tail)], iv)
            pltpu.async_copy(table_hbm.at[iv], rv, sem).wait()
            pltpu.sync_copy(rv, out_hbm.at[pl.ds(start, tail)])

    return k(table, idx)


def _winner_body(sp_ref, gu_ref, out_ref):
    s0 = jnp.log(jnp.maximum(sp_ref[0], 1e-20)) + gu_ref[0]
    s1 = jnp.log(jnp.maximum(sp_ref[1], 1e-20)) + gu_ref[1]
    s2 = jnp.log(jnp.maximum(sp_ref[2], 1e-20)) + gu_ref[2]
    amax = jnp.where(
        (s0 >= s1) & (s0 >= s2),
        jnp.int32(0),
        jnp.where(s1 >= s2, jnp.int32(1), jnp.int32(2)),
    )
    pid = pl.program_id(0)
    row = lax.broadcasted_iota(jnp.int32, (_BR, _COLS), 0)
    col = lax.broadcasted_iota(jnp.int32, (_BR, _COLS), 1)
    gidx = (pid * _BR + row) * _COLS + col
    out_ref[...] = amax * _G + gidx


def _select_winners(sp_t, gu_t):
    return pl.pallas_call(
        _winner_body,
        grid=(_ROWS // _BR,),
        in_specs=[
            pl.BlockSpec(
                (3, _BR, _COLS), lambda i: (jnp.int32(0), i, jnp.int32(0))
            ),
            pl.BlockSpec(
                (3, _BR, _COLS), lambda i: (jnp.int32(0), i, jnp.int32(0))
            ),
        ],
        out_specs=pl.BlockSpec((_BR, _COLS), lambda i: (i, jnp.int32(0))),
        out_shape=jax.ShapeDtypeStruct((_ROWS, _COLS), jnp.int32),
    )(sp_t, gu_t)


def kernel(probs, groups):
    probs = probs.astype(jnp.float32)
    g32 = groups.astype(jnp.int32)

    si = jnp.argsort(g32).astype(jnp.int32)
    idx_t = si.reshape(_G, _EPG).transpose(1, 0).reshape(-1)

    sp_t = _sc_gather(probs, idx_t).reshape(_EPG, _ROWS, _COLS)

    gu = jax.random.gumbel(jax.random.key(42), (_G, _EPG), dtype=jnp.float32)
    gu_t = gu.transpose(1, 0).reshape(_EPG, _ROWS, _COLS)

    sel = _select_winners(sp_t, gu_t).reshape(-1)

    wpos = _sc_gather(idx_t, sel)
    sampled = jnp.sort(wpos)

    sprobs = _sc_gather(probs, sampled)

    return sampled.astype(jnp.int64), sprobs
